# baseline (speedup 1.0000x reference)
import jax
import jax.numpy as jnp
from jax import lax
from jax.experimental import pallas as pl
from jax.experimental.pallas import tpu as pltpu
from jax.experimental.pallas import tpu_sc as plsc

RES = 128
THRESH = 0.01
N = 2097152
N_CELLS = RES * RES * RES
PAD = 8192
TOT = N_CELLS + PAD

NUM_TILES = 16
NPT = N // NUM_TILES
CHUNK = 8192
ROWS = CHUNK // 128
NCHUNK = NPT // CHUNK


def _body(coords_ref, dens_ref, grid_ref, out_ref,
          xv, yv, zv, dv, idxbuf, ones, sem_in, sem_scat):
    sid = lax.axis_index("s")

    for i in range(8):
        ones[pl.ds(i * 16, 16)] = jnp.full((16,), 1.0, jnp.float32)

    slab = N_CELLS // NUM_TILES
    pltpu.sync_copy(grid_ref.at[pl.ds(sid * slab, slab)],
                    out_ref.at[pl.ds(sid * slab, slab)])
    padslab = PAD // NUM_TILES
    pltpu.sync_copy(grid_ref.at[pl.ds(sid * padslab, padslab)],
                    out_ref.at[pl.ds(N_CELLS + sid * padslab, padslab)])
    plsc.subcore_barrier()

    lane = lax.iota(jnp.int32, 16)

    def start_in(k):
        b = k % 2
        base = sid * NPT + k * CHUNK
        pltpu.async_copy(coords_ref.at[pl.ds(base, CHUNK)],
                         xv.at[b], sem_in.at[b])
        pltpu.async_copy(coords_ref.at[pl.ds(N + base, CHUNK)],
                         yv.at[b], sem_in.at[b])
        pltpu.async_copy(coords_ref.at[pl.ds(2 * N + base, CHUNK)],
                         zv.at[b], sem_in.at[b])
        pltpu.async_copy(dens_ref.at[pl.ds(base, CHUNK)],
                         dv.at[b], sem_in.at[b])

    def wait_in(k):
        b = k % 2
        pltpu.make_async_copy(coords_ref.at[pl.ds(0, CHUNK)],
                              xv.at[b], sem_in.at[b]).wait()
        pltpu.make_async_copy(coords_ref.at[pl.ds(0, CHUNK)],
                              yv.at[b], sem_in.at[b]).wait()
        pltpu.make_async_copy(coords_ref.at[pl.ds(0, CHUNK)],
                              zv.at[b], sem_in.at[b]).wait()
        pltpu.make_async_copy(dens_ref.at[pl.ds(0, CHUNK)],
                              dv.at[b], sem_in.at[b]).wait()

    def compute(k):
        b = k % 2

        @pl.loop(0, ROWS)
        def _row(r):
            for g in range(8):
                off = r * 128 + g * 16
                pid = off + lane
                x = xv[b, pl.ds(off, 16)]
                y = yv[b, pl.ds(off, 16)]
                z = zv[b, pl.ds(off, 16)]
                ix = jnp.clip((x * 127.0).astype(jnp.int32), 0, RES - 1)
                iy = jnp.clip((y * 127.0).astype(jnp.int32), 0, RES - 1)
                iz = jnp.clip((z * 127.0).astype(jnp.int32), 0, RES - 1)
                lin = (ix * RES + iy) * RES + iz
                d = dv[b, pl.ds(off, 16)]
                trash = N_CELLS + (pid & (PAD - 1))
                idxbuf[b, r, pl.ds(g * 16, 16)] = jnp.where(
                    d > THRESH, lin, trash)

    def fire_scat(k):
        b = k % 2

        @pl.loop(0, ROWS)
        def _scat(r):
            pltpu.async_copy(ones, out_ref.at[idxbuf.at[b, r]], sem_scat.at[b])

    def drain_scat(k):
        b = k % 2

        @pl.loop(0, ROWS)
        def _drain(r):
            pltpu.make_async_copy(ones, out_ref.at[idxbuf.at[b, r]],
                                  sem_scat.at[b]).wait()

    start_in(0)
    for k in range(NCHUNK):
        wait_in(k)
        if k + 1 < NCHUNK:
            start_in(k + 1)
        if k >= 2:
            drain_scat(k - 2)
        compute(k)
        fire_scat(k)
    drain_scat(NCHUNK - 2)
    drain_scat(NCHUNK - 1)


_mesh = plsc.VectorSubcoreMesh(
    core_axis_name="c", subcore_axis_name="s", num_cores=1)

_scatter = pl.kernel(
    _body,
    out_type=jax.ShapeDtypeStruct((TOT,), jnp.float32),
    mesh=_mesh,
    scratch_types=[
        pltpu.VMEM((2, CHUNK), jnp.float32),
        pltpu.VMEM((2, CHUNK), jnp.float32),
        pltpu.VMEM((2, CHUNK), jnp.float32),
        pltpu.VMEM((2, CHUNK), jnp.float32),
        pltpu.VMEM((2, ROWS, 128), jnp.int32),
        pltpu.VMEM((128,), jnp.float32),
        pltpu.SemaphoreType.DMA((2,)),
        pltpu.SemaphoreType.DMA((2,)),
    ],
)


@jax.jit
def kernel(coords, densities, grid):
    coords_t = coords.T.reshape(-1)
    out = _scatter(coords_t, densities, grid.reshape(-1))
    return out[:N_CELLS].reshape(RES, RES, RES)
